# baseline (speedup 1.0000x reference)
import functools

import jax
import jax.numpy as jnp
from jax import lax
from jax.experimental import pallas as pl
from jax.experimental.pallas import tpu as pltpu

_C = 1000
_M = 32
_D = 1024
_B = 4096
_BBLK = 512
_NSTEPS = _B // _BBLK
_QS = 32768.0
_KS = 262144


def _kernel(logits_ref, emt_ref, feats_ref, mem_ref, out_ref,
            key_v, slot_v, worst_v, scr_v, dowin_s, slot_s, cnt_s,
            buf, sem_a, sem_b, sem_c):
    del mem_ref
    i = pl.program_id(0)

    @pl.when(i == 0)
    def _init():
        key_v[...] = jnp.zeros_like(key_v)
        emt = emt_ref[...]
        w = jnp.max(emt, axis=1, keepdims=True)
        sub = lax.broadcasted_iota(jnp.int32, (_C, _M), 1)
        s = jnp.min(jnp.where(emt == w, sub, _M), axis=1, keepdims=True)
        slot_v[...] = jnp.transpose(s)
        worst_v[...] = jnp.transpose(w)

    l = logits_ref[...]
    m = jnp.max(l, axis=1, keepdims=True)
    d = l - m
    e = jnp.exp(d)
    z = jnp.sum(e, axis=1, keepdims=True)
    s1 = jnp.sum(e * d, axis=1, keepdims=True)
    ent = jnp.log(z) - s1 / z
    q = jnp.clip((ent * _QS + 0.5).astype(jnp.int32), 0, _KS - 1)
    lane = lax.broadcasted_iota(jnp.int32, (_BBLK, _C), 1)
    t = jnp.where(l == m, lane, _C)
    pseudo = jnp.min(t, axis=1, keepdims=True)
    row = lax.broadcasted_iota(jnp.int32, (_BBLK, 1), 0)
    keyrow = (i * _BBLK + row + 1) * _KS + q
    kblk = jnp.max(jnp.where(t == pseudo, keyrow, 0), axis=0, keepdims=True)
    key_v[...] = jnp.maximum(key_v[...], kblk)

    @pl.when(i == _NSTEPS - 1)
    def _fin():
        key = key_v[...]
        wplus = lax.shift_right_logical(key, 18)
        entwin = (key - wplus * _KS).astype(jnp.float32) * (1.0 / _QS)
        do = (wplus > 0) & (entwin < worst_v[...])
        dowin = jnp.where(do, wplus, 0)
        key_v[...] = dowin
        scr_v[...] = jnp.sum(dowin, keepdims=True) * jnp.ones_like(scr_v)
        cp_a = pltpu.make_async_copy(key_v, dowin_s, sem_a)
        cp_b = pltpu.make_async_copy(slot_v, slot_s, sem_b)
        cp_c = pltpu.make_async_copy(scr_v, cnt_s, sem_c)
        cp_a.start()
        cp_b.start()
        cp_c.start()
        cp_a.wait()
        cp_b.wait()
        cp_c.wait()

        @pl.when(cnt_s[0, 0] > 0)
        def _any():
            def body(c, carry):
                wp = dowin_s[0, c]

                @pl.when(wp > 0)
                def _write():
                    b = wp - 1
                    s = slot_s[0, c]
                    cp = pltpu.make_async_copy(
                        feats_ref.at[pl.ds(b, 1), :], buf, sem_a)
                    cp.start()
                    cp.wait()
                    r = buf[...]
                    buf[...] = r * lax.rsqrt(jnp.sum(r * r, keepdims=True))
                    cp2 = pltpu.make_async_copy(
                        buf, out_ref.at[c, pl.ds(s, 1), :], sem_b)
                    cp2.start()
                    cp2.wait()
                return carry
            lax.fori_loop(0, _C, body, 0)


@functools.partial(jax.jit, static_argnames=("interpret",))
def _impl(feature_memory, entropy_memory, logits, image_features_global,
          interpret=False):
    new_mem = pl.pallas_call(
        _kernel,
        grid=(_NSTEPS,),
        in_specs=[
            pl.BlockSpec((_BBLK, _C), lambda i: (i, 0)),
            pl.BlockSpec((_C, _M), lambda i: (0, 0)),
            pl.BlockSpec(memory_space=pltpu.MemorySpace.HBM),
            pl.BlockSpec(memory_space=pltpu.MemorySpace.HBM),
        ],
        out_specs=pl.BlockSpec(memory_space=pltpu.MemorySpace.HBM),
        out_shape=jax.ShapeDtypeStruct((_C, _M, _D), jnp.float32),
        scratch_shapes=[
            pltpu.VMEM((1, _C), jnp.int32),
            pltpu.VMEM((1, _C), jnp.int32),
            pltpu.VMEM((1, _C), jnp.float32),
            pltpu.VMEM((1, _C), jnp.int32),
            pltpu.SMEM((1, _C), jnp.int32),
            pltpu.SMEM((1, _C), jnp.int32),
            pltpu.SMEM((1, _C), jnp.int32),
            pltpu.VMEM((1, _D), jnp.float32),
            pltpu.SemaphoreType.DMA,
            pltpu.SemaphoreType.DMA,
            pltpu.SemaphoreType.DMA,
        ],
        input_output_aliases={3: 0},
        interpret=interpret,
    )(logits, entropy_memory, image_features_global, feature_memory)
    return new_mem


def kernel(feature_memory, entropy_memory, logits, image_features_global):
    return _impl(feature_memory, entropy_memory, logits,
                 image_features_global)


# speedup vs baseline: 1.0178x; 1.0178x over previous
import functools

import jax
import jax.numpy as jnp
from jax import lax
from jax.experimental import pallas as pl
from jax.experimental.pallas import tpu as pltpu

_C = 1000
_M = 32
_D = 1024
_B = 4096
_BBLK = 512
_NSTEPS = _B // _BBLK
_QS = 32768.0
_KS = 262144


def _kernel(logits_ref, emt_ref, feats_ref, mem_ref, out_ref,
            key_v, slot_v, worst_v, scr_v, dowin_s, slot_s, cnt_s,
            buf, sem_a, sem_b, sem_c):
    del mem_ref
    i = pl.program_id(0)

    @pl.when(i == 0)
    def _init():
        key_v[...] = jnp.zeros_like(key_v)
        emt = emt_ref[...]
        w = jnp.max(emt, axis=0, keepdims=True)
        sub = lax.broadcasted_iota(jnp.int32, (_M, _C), 0)
        slot_v[...] = jnp.min(jnp.where(emt == w, sub, _M), axis=0,
                              keepdims=True)
        worst_v[...] = w
        pltpu.make_async_copy(slot_v, slot_s, sem_b).start()

    l = logits_ref[...]
    m = jnp.max(l, axis=1, keepdims=True)
    d = l - m
    e = jnp.exp(d)
    z = jnp.sum(e, axis=1, keepdims=True)
    s1 = jnp.sum(e * d, axis=1, keepdims=True)
    ent = jnp.log(z) - s1 / z
    q = jnp.clip((ent * _QS + 0.5).astype(jnp.int32), 0, _KS - 1)
    lane = lax.broadcasted_iota(jnp.int32, (_BBLK, _C), 1)
    t = jnp.where(l == m, lane, _C)
    pseudo = jnp.min(t, axis=1, keepdims=True)
    row = lax.broadcasted_iota(jnp.int32, (_BBLK, 1), 0)
    keyrow = (i * _BBLK + row + 1) * _KS + q
    kblk = jnp.max(jnp.where(t == pseudo, keyrow, 0), axis=0, keepdims=True)
    key_v[...] = jnp.maximum(key_v[...], kblk)

    @pl.when(i == _NSTEPS - 1)
    def _fin():
        key = key_v[...]
        wplus = lax.shift_right_logical(key, 18)
        entwin = (key - wplus * _KS).astype(jnp.float32) * (1.0 / _QS)
        do = (wplus > 0) & (entwin < worst_v[...])
        dowin = jnp.where(do, wplus, 0)
        key_v[...] = dowin
        scr_v[...] = jnp.sum(dowin, keepdims=True) * jnp.ones_like(scr_v)
        cp_a = pltpu.make_async_copy(key_v, dowin_s, sem_a)
        cp_c = pltpu.make_async_copy(scr_v, cnt_s, sem_c)
        cp_a.start()
        cp_c.start()
        pltpu.make_async_copy(slot_v, slot_s, sem_b).wait()
        cp_a.wait()
        cp_c.wait()

        @pl.when(cnt_s[0, 0] > 0)
        def _any():
            def body(c, carry):
                wp = dowin_s[0, c]

                @pl.when(wp > 0)
                def _write():
                    b = wp - 1
                    s = slot_s[0, c]
                    cp = pltpu.make_async_copy(
                        feats_ref.at[pl.ds(b, 1), :], buf, sem_a)
                    cp.start()
                    cp.wait()
                    r = buf[...]
                    buf[...] = r * lax.rsqrt(jnp.sum(r * r, keepdims=True))
                    cp2 = pltpu.make_async_copy(
                        buf, out_ref.at[c, pl.ds(s, 1), :], sem_b)
                    cp2.start()
                    cp2.wait()
                return carry
            lax.fori_loop(0, _C, body, 0)


@functools.partial(jax.jit, static_argnames=("interpret",))
def _impl(feature_memory, entropy_memory, logits, image_features_global,
          interpret=False):
    emt = entropy_memory.T
    new_mem = pl.pallas_call(
        _kernel,
        grid=(_NSTEPS,),
        in_specs=[
            pl.BlockSpec((_BBLK, _C), lambda i: (i, 0)),
            pl.BlockSpec((_M, _C), lambda i: (0, 0)),
            pl.BlockSpec(memory_space=pltpu.MemorySpace.HBM),
            pl.BlockSpec(memory_space=pltpu.MemorySpace.HBM),
        ],
        out_specs=pl.BlockSpec(memory_space=pltpu.MemorySpace.HBM),
        out_shape=jax.ShapeDtypeStruct((_C, _M, _D), jnp.float32),
        scratch_shapes=[
            pltpu.VMEM((1, _C), jnp.int32),
            pltpu.VMEM((1, _C), jnp.int32),
            pltpu.VMEM((1, _C), jnp.float32),
            pltpu.VMEM((1, _C), jnp.int32),
            pltpu.SMEM((1, _C), jnp.int32),
            pltpu.SMEM((1, _C), jnp.int32),
            pltpu.SMEM((1, _C), jnp.int32),
            pltpu.VMEM((1, _D), jnp.float32),
            pltpu.SemaphoreType.DMA,
            pltpu.SemaphoreType.DMA,
            pltpu.SemaphoreType.DMA,
        ],
        input_output_aliases={3: 0},
        interpret=interpret,
    )(logits, emt, image_features_global, feature_memory)
    return new_mem


def kernel(feature_memory, entropy_memory, logits, image_features_global):
    return _impl(feature_memory, entropy_memory, logits,
                 image_features_global)
